# baseline (speedup 1.0000x reference)
import jax
import jax.numpy as jnp
from jax import lax
from jax.experimental import pallas as pl
from jax.experimental.pallas import tpu as pltpu
from jax.experimental.pallas import tpu_sc as plsc

_VOCAB = 1000000
_SEQ = 200
_D = 64
_BATCH = 4096

_NC = 2
_NS = 16
_NW = _NC * _NS

_BPW = _BATCH // _NW
_NBUF = 2
_NOUTER = _SEQ // _NBUF
_LANES = 16
_PAIR_W = 2 * _D
_KG = _BPW // _LANES

_BLK = 256
_NFULL = _VOCAB // _BLK
_BPW_TR = _NFULL // _NW
_TR_EXTRA = _NFULL - _BPW_TR * _NW


def _tr_transpose_block(ibuf, obuf, iota, half, qbit, n_r0):
    @plsc.parallel_loop(0, _D, 1, unroll=4)
    def do_t(t):
        dvec = ((iota + t) & (_LANES - 1)) + (t & ~(_LANES - 1))
        colv = qbit + dvec
        for r0 in range(0, n_r0 * 8, 8):
            vals = plsc.load_gather(ibuf, [dvec, iota + 2 * r0])
            plsc.store_scatter(obuf, [half + r0, colv], vals)


def _tr_body(tabT_hbm, tailp_hbm, tab2_hbm, ibufs, obufs, isems, osems):
    wid = lax.axis_index("s") * _NC + lax.axis_index("c")
    b0 = wid * _BPW_TR
    iota = lax.iota(jnp.int32, _LANES)
    half = lax.shift_right_logical(iota, 1)
    qbit = (iota & 1) * _D

    def in_slice(g):
        off = pl.multiple_of((b0 + g) * _BLK, _BLK)
        return tabT_hbm.at[:, pl.ds(off, _BLK)]

    def fire_in(g, b):
        pltpu.async_copy(in_slice(g), ibufs[b], isems[b])

    def wait_in(g, b):
        pltpu.make_async_copy(in_slice(g), ibufs[b], isems[b]).wait()

    def out_slice(g):
        return tab2_hbm.at[pl.ds((b0 + g) * (_BLK // 2), _BLK // 2)]

    for b in range(_NBUF):
        fire_in(b, b)

    def outer(o, carry):
        for b in range(_NBUF):
            g = o * _NBUF + b
            wait_in(g, b)

            @pl.when(o > 0)
            def _():
                pltpu.make_async_copy(obufs[b], out_slice(g), osems[b]).wait()

            _tr_transpose_block(ibufs[b], obufs[b], iota, half, qbit,
                                _BLK // _LANES)
            pltpu.async_copy(obufs[b], out_slice(g), osems[b])

            @pl.when(g + _NBUF < _BPW_TR)
            def _():
                fire_in(g + _NBUF, b)

        return carry

    lax.fori_loop(0, _BPW_TR // _NBUF, outer, 0)
    for b in range(_NBUF):
        g = _BPW_TR - _NBUF + b
        pltpu.make_async_copy(obufs[b], out_slice(g), osems[b]).wait()

    @pl.when(wid < _TR_EXTRA)
    def _():
        blk = _BPW_TR * _NW + wid
        pltpu.sync_copy(tabT_hbm.at[:, pl.ds(blk * _BLK, _BLK)], ibufs[0])
        _tr_transpose_block(ibufs[0], obufs[0], iota, half, qbit,
                            _BLK // _LANES)
        pltpu.sync_copy(obufs[0],
                        tab2_hbm.at[pl.ds(blk * (_BLK // 2), _BLK // 2)])

    @pl.when(wid == _TR_EXTRA)
    def _():
        pltpu.sync_copy(tailp_hbm.at[pl.ds(_D // 2, _D // 2)],
                        tab2_hbm.at[pl.ds((_VOCAB - _D) // 2, _D // 2)])


def _tr_kernel(tabT_hbm, tailp_hbm, tab2_hbm, i0_, i1_, o0_, o1_,
               is0, is1, os0, os1):
    _tr_body(tabT_hbm, tailp_hbm, tab2_hbm, [i0_, i1_], [o0_, o1_],
             [is0, is1], [os0, os1])


def _gather_body(tab2_hbm, idxT_hbm, pe_hbm, outT_hbm,
                 idx_all, pe_v, ihs, gbufs, tbufs, gsems, osems):
    wid = lax.axis_index("s") * _NC + lax.axis_index("c")
    b0 = wid * _BPW

    pltpu.sync_copy(idxT_hbm.at[:, pl.ds(b0, _BPW)], idx_all)
    pltpu.sync_copy(pe_hbm, pe_v)

    def fire_gather(s, b):
        for k in range(_KG):
            sl = pl.ds(k * _LANES, _LANES)
            ihs[b][sl] = lax.shift_right_logical(idx_all[s, sl], 1)
        pltpu.async_copy(tab2_hbm.at[ihs[b]], gbufs[b], gsems[b])

    def wait_gather(b):
        pltpu.make_async_copy(tab2_hbm.at[ihs[b]], gbufs[b], gsems[b]).wait()

    def out_slice(s):
        return outT_hbm.at[s, :, pl.ds(b0, _BPW)]

    for b in range(_NBUF):
        fire_gather(b, b)

    def outer(o, carry):
        for b in range(_NBUF):
            s = o * _NBUF + b
            wait_gather(b)

            @pl.when(o > 0)
            def _():
                pltpu.make_async_copy(tbufs[b], out_slice(s),
                                      osems[b]).wait()

            gbuf, tbuf = gbufs[b], tbufs[b]
            iota = lax.iota(jnp.int32, _LANES)
            colbase = []
            for k in range(_KG):
                sl = pl.ds(k * _LANES, _LANES)
                colbase.append((idx_all[s, sl] & 1) * _D)

            @plsc.parallel_loop(0, _D, 1, unroll=4)
            def do_t(t):
                dvec = ((iota + t) & (_LANES - 1)) + (t & ~(_LANES - 1))
                pe_vec = plsc.load_gather(pe_v, [s * _D + dvec])
                for k in range(_KG):
                    rows_k = iota + (k * _LANES)
                    vals = plsc.load_gather(gbuf, [rows_k, colbase[k] + dvec])
                    plsc.store_scatter(tbuf, [dvec, rows_k], vals + pe_vec)

            pltpu.async_copy(tbuf, out_slice(s), osems[b])

            @pl.when(s + _NBUF < _SEQ)
            def _():
                fire_gather(s + _NBUF, b)

        return carry

    lax.fori_loop(0, _NOUTER, outer, 0)

    for b in range(_NBUF):
        s = (_NOUTER - 1) * _NBUF + b
        pltpu.make_async_copy(tbufs[b], out_slice(s), osems[b]).wait()


def _gather_kernel(tab2_hbm, idxT_hbm, pe_hbm, outT_hbm,
                   idx_all, pe_v, i0_, i1_, g0, g1, t0, t1,
                   gs0, gs1, os0, os1):
    _gather_body(tab2_hbm, idxT_hbm, pe_hbm, outT_hbm, idx_all, pe_v,
                 [i0_, i1_], [g0, g1], [t0, t1], [gs0, gs1], [os0, os1])


@jax.jit
def _sc_embed(tabT, tailp, idxT, pe_flat):
    mesh = plsc.VectorSubcoreMesh(core_axis_name="c", subcore_axis_name="s")
    tab2 = pl.kernel(
        _tr_kernel,
        out_type=jax.ShapeDtypeStruct((_VOCAB // 2, _PAIR_W), jnp.float32),
        mesh=mesh,
        scratch_types=(
            [pltpu.VMEM((_D, _BLK), jnp.float32) for _ in range(_NBUF)]
            + [pltpu.VMEM((_BLK // 2, _PAIR_W), jnp.float32)
               for _ in range(_NBUF)]
            + [pltpu.SemaphoreType.DMA for _ in range(2 * _NBUF)]
        ),
        compiler_params=pltpu.CompilerParams(needs_layout_passes=False),
    )(tabT, tailp)
    return pl.kernel(
        _gather_kernel,
        out_type=jax.ShapeDtypeStruct((_SEQ, _D, _BATCH), jnp.float32),
        mesh=mesh,
        scratch_types=(
            [pltpu.VMEM((_SEQ, _BPW), jnp.int32),
             pltpu.VMEM((_SEQ * _D,), jnp.float32)]
            + [pltpu.VMEM((_BPW,), jnp.int32) for _ in range(_NBUF)]
            + [pltpu.VMEM((_BPW, _PAIR_W), jnp.float32) for _ in range(_NBUF)]
            + [pltpu.VMEM((_D, _BPW), jnp.float32) for _ in range(_NBUF)]
            + [pltpu.SemaphoreType.DMA for _ in range(2 * _NBUF)]
        ),
        compiler_params=pltpu.CompilerParams(needs_layout_passes=False),
    )(tab2, idxT, pe_flat)


def kernel(indices, table, pos_enc):
    tabT = table.T
    tailp = table[-2 * _D:].reshape(_D, _PAIR_W)
    idxT = indices.T.astype(jnp.int32)
    pe_flat = pos_enc.reshape(-1)
    outT = _sc_embed(tabT, tailp, idxT, pe_flat)
    return outT.transpose(2, 0, 1)


# speedup vs baseline: 1.0714x; 1.0714x over previous
import jax
import jax.numpy as jnp
from jax import lax
from jax.experimental import pallas as pl
from jax.experimental.pallas import tpu as pltpu
from jax.experimental.pallas import tpu_sc as plsc

_VOCAB = 1000000
_SEQ = 200
_D = 64
_BATCH = 4096

_NC = 2
_NS = 16
_NW = _NC * _NS

_BPW = _BATCH // _NW
_NBUF = 2
_NOUTER = _SEQ // _NBUF
_LANES = 16
_PAIR_W = 2 * _D
_KG = _BPW // _LANES

_BLK = 128
_NFULL = _VOCAB // _BLK
_BPW_TR = _NFULL // _NW
_TR_EXTRA = _NFULL - _BPW_TR * _NW


def _tr_transpose_block(ibuf, obuf, iota, half, qbit, n_r0):
    @plsc.parallel_loop(0, _D, 1, unroll=4)
    def do_t(t):
        dvec = ((iota + t) & (_LANES - 1)) + (t & ~(_LANES - 1))
        colv = qbit + dvec
        for r0 in range(0, n_r0 * 8, 8):
            vals = plsc.load_gather(ibuf, [dvec, iota + 2 * r0])
            plsc.store_scatter(obuf, [half + r0, colv], vals)


def _tr_body(tabT_hbm, tailp_hbm, tab2_hbm, ibufs, obufs, isems, osems):
    wid = lax.axis_index("s") * _NC + lax.axis_index("c")
    b0 = wid * _BPW_TR
    iota = lax.iota(jnp.int32, _LANES)
    half = lax.shift_right_logical(iota, 1)
    qbit = (iota & 1) * _D

    def in_slice(g):
        off = pl.multiple_of((b0 + g) * _BLK, _BLK)
        return tabT_hbm.at[:, pl.ds(off, _BLK)]

    def fire_in(g, b):
        pltpu.async_copy(in_slice(g), ibufs[b], isems[b])

    def wait_in(g, b):
        pltpu.make_async_copy(in_slice(g), ibufs[b], isems[b]).wait()

    def out_slice(g):
        return tab2_hbm.at[pl.ds((b0 + g) * (_BLK // 2), _BLK // 2)]

    for b in range(_NBUF):
        fire_in(b, b)

    def outer(o, carry):
        for b in range(_NBUF):
            g = o * _NBUF + b
            wait_in(g, b)

            @pl.when(o > 0)
            def _():
                pltpu.make_async_copy(obufs[b], out_slice(g), osems[b]).wait()

            _tr_transpose_block(ibufs[b], obufs[b], iota, half, qbit,
                                _BLK // _LANES)
            pltpu.async_copy(obufs[b], out_slice(g), osems[b])

            @pl.when(g + _NBUF < _BPW_TR)
            def _():
                fire_in(g + _NBUF, b)

        return carry

    lax.fori_loop(0, _BPW_TR // _NBUF, outer, 0)
    for b in range(_NBUF):
        g = _BPW_TR - _NBUF + b
        pltpu.make_async_copy(obufs[b], out_slice(g), osems[b]).wait()

    @pl.when(wid < _TR_EXTRA)
    def _():
        blk = _BPW_TR * _NW + wid
        pltpu.sync_copy(tabT_hbm.at[:, pl.ds(blk * _BLK, _BLK)], ibufs[0])
        _tr_transpose_block(ibufs[0], obufs[0], iota, half, qbit,
                            _BLK // _LANES)
        pltpu.sync_copy(obufs[0],
                        tab2_hbm.at[pl.ds(blk * (_BLK // 2), _BLK // 2)])

    @pl.when(wid == _TR_EXTRA)
    def _():
        pltpu.sync_copy(tailp_hbm.at[pl.ds(_D // 2, _D // 2)],
                        tab2_hbm.at[pl.ds((_VOCAB - _D) // 2, _D // 2)])


def _tr_kernel(tabT_hbm, tailp_hbm, tab2_hbm, i0_, i1_, o0_, o1_,
               is0, is1, os0, os1):
    _tr_body(tabT_hbm, tailp_hbm, tab2_hbm, [i0_, i1_], [o0_, o1_],
             [is0, is1], [os0, os1])


def _gather_body(tab2_hbm, idxT_hbm, pe_hbm, outT_hbm,
                 idx_all, pe_v, ihs, gbufs, tbufs, gsems, osems):
    wid = lax.axis_index("s") * _NC + lax.axis_index("c")
    b0 = wid * _BPW

    pltpu.sync_copy(idxT_hbm.at[:, pl.ds(b0, _BPW)], idx_all)
    pltpu.sync_copy(pe_hbm, pe_v)

    def fire_gather(s, b):
        for k in range(_KG):
            sl = pl.ds(k * _LANES, _LANES)
            ihs[b][sl] = lax.shift_right_logical(idx_all[s, sl], 1)
        pltpu.async_copy(tab2_hbm.at[ihs[b]], gbufs[b], gsems[b])

    def wait_gather(b):
        pltpu.make_async_copy(tab2_hbm.at[ihs[b]], gbufs[b], gsems[b]).wait()

    def out_slice(s):
        return outT_hbm.at[s, :, pl.ds(b0, _BPW)]

    for b in range(_NBUF):
        fire_gather(b, b)

    def outer(o, carry):
        for b in range(_NBUF):
            s = o * _NBUF + b
            wait_gather(b)

            @pl.when(o > 0)
            def _():
                pltpu.make_async_copy(tbufs[b], out_slice(s),
                                      osems[b]).wait()

            gbuf, tbuf = gbufs[b], tbufs[b]
            iota = lax.iota(jnp.int32, _LANES)
            colbase = []
            for k in range(_KG):
                sl = pl.ds(k * _LANES, _LANES)
                colbase.append((idx_all[s, sl] & 1) * _D)

            @plsc.parallel_loop(0, _D, 1, unroll=4)
            def do_t(t):
                dvec = ((iota + t) & (_LANES - 1)) + (t & ~(_LANES - 1))
                pe_vec = plsc.load_gather(pe_v, [s * _D + dvec])
                for k in range(_KG):
                    rows_k = iota + (k * _LANES)
                    vals = plsc.load_gather(gbuf, [rows_k, colbase[k] + dvec])
                    plsc.store_scatter(tbuf, [dvec, rows_k], vals + pe_vec)

            pltpu.async_copy(tbuf, out_slice(s), osems[b])

            @pl.when(s + _NBUF < _SEQ)
            def _():
                fire_gather(s + _NBUF, b)

        return carry

    lax.fori_loop(0, _NOUTER, outer, 0)

    for b in range(_NBUF):
        s = (_NOUTER - 1) * _NBUF + b
        pltpu.make_async_copy(tbufs[b], out_slice(s), osems[b]).wait()


def _gather_kernel(tab2_hbm, idxT_hbm, pe_hbm, outT_hbm,
                   idx_all, pe_v, i0_, i1_, g0, g1, t0, t1,
                   gs0, gs1, os0, os1):
    _gather_body(tab2_hbm, idxT_hbm, pe_hbm, outT_hbm, idx_all, pe_v,
                 [i0_, i1_], [g0, g1], [t0, t1], [gs0, gs1], [os0, os1])


@jax.jit
def _sc_embed(tabT, tailp, idxT, pe_flat):
    mesh = plsc.VectorSubcoreMesh(core_axis_name="c", subcore_axis_name="s")
    tab2 = pl.kernel(
        _tr_kernel,
        out_type=jax.ShapeDtypeStruct((_VOCAB // 2, _PAIR_W), jnp.float32),
        mesh=mesh,
        scratch_types=(
            [pltpu.VMEM((_D, _BLK), jnp.float32) for _ in range(_NBUF)]
            + [pltpu.VMEM((_BLK // 2, _PAIR_W), jnp.float32)
               for _ in range(_NBUF)]
            + [pltpu.SemaphoreType.DMA for _ in range(2 * _NBUF)]
        ),
        compiler_params=pltpu.CompilerParams(needs_layout_passes=False),
    )(tabT, tailp)
    return pl.kernel(
        _gather_kernel,
        out_type=jax.ShapeDtypeStruct((_SEQ, _D, _BATCH), jnp.float32),
        mesh=mesh,
        scratch_types=(
            [pltpu.VMEM((_SEQ, _BPW), jnp.int32),
             pltpu.VMEM((_SEQ * _D,), jnp.float32)]
            + [pltpu.VMEM((_BPW,), jnp.int32) for _ in range(_NBUF)]
            + [pltpu.VMEM((_BPW, _PAIR_W), jnp.float32) for _ in range(_NBUF)]
            + [pltpu.VMEM((_D, _BPW), jnp.float32) for _ in range(_NBUF)]
            + [pltpu.SemaphoreType.DMA for _ in range(2 * _NBUF)]
        ),
        compiler_params=pltpu.CompilerParams(needs_layout_passes=False),
    )(tab2, idxT, pe_flat)


def kernel(indices, table, pos_enc):
    tabT = table.T
    tailp = table[-2 * _D:].reshape(_D, _PAIR_W)
    idxT = indices.T.astype(jnp.int32)
    pe_flat = pos_enc.reshape(-1)
    outT = _sc_embed(tabT, tailp, idxT, pe_flat)
    return outT.transpose(2, 0, 1)


# speedup vs baseline: 1.1523x; 1.0755x over previous
import jax
import jax.numpy as jnp
from jax import lax
from jax.experimental import pallas as pl
from jax.experimental.pallas import tpu as pltpu
from jax.experimental.pallas import tpu_sc as plsc

_VOCAB = 1000000
_SEQ = 200
_D = 64
_BATCH = 4096

_NC = 2
_NS = 16
_NW = _NC * _NS

_BPW = _BATCH // _NW
_NBUF = 2
_NOUTER = _SEQ // _NBUF
_LANES = 16
_PAIR_W = 2 * _D
_KG = _BPW // _LANES

_NBUF_TR = 4
_BLK = 128
_NFULL = _VOCAB // _BLK
_BPW_TR = _NFULL // _NW
_TR_EXTRA = _NFULL - _BPW_TR * _NW


def _tr_transpose_block(ibuf, obuf, iota, half, qbit, n_r0):
    @plsc.parallel_loop(0, _D, 1, unroll=4)
    def do_t(t):
        dvec = ((iota + t) & (_LANES - 1)) + (t & ~(_LANES - 1))
        colv = qbit + dvec
        for r0 in range(0, n_r0 * 8, 8):
            vals = plsc.load_gather(ibuf, [dvec, iota + 2 * r0])
            plsc.store_scatter(obuf, [half + r0, colv], vals)


def _tr_body(tabT_hbm, tailp_hbm, tab2_hbm, ibufs, obufs, isems, osems):
    wid = lax.axis_index("s") * _NC + lax.axis_index("c")
    b0 = wid * _BPW_TR
    iota = lax.iota(jnp.int32, _LANES)
    half = lax.shift_right_logical(iota, 1)
    qbit = (iota & 1) * _D

    def in_slice(g):
        off = pl.multiple_of((b0 + g) * _BLK, _BLK)
        return tabT_hbm.at[:, pl.ds(off, _BLK)]

    def fire_in(g, b):
        pltpu.async_copy(in_slice(g), ibufs[b], isems[b])

    def wait_in(g, b):
        pltpu.make_async_copy(in_slice(g), ibufs[b], isems[b]).wait()

    def out_slice(g):
        return tab2_hbm.at[pl.ds((b0 + g) * (_BLK // 2), _BLK // 2)]

    for b in range(_NBUF_TR):
        fire_in(b, b)

    def outer(o, carry):
        for b in range(_NBUF_TR):
            g = o * _NBUF_TR + b
            wait_in(g, b)

            @pl.when(o > 0)
            def _():
                pltpu.make_async_copy(obufs[b], out_slice(g), osems[b]).wait()

            _tr_transpose_block(ibufs[b], obufs[b], iota, half, qbit,
                                _BLK // _LANES)
            pltpu.async_copy(obufs[b], out_slice(g), osems[b])

            @pl.when(g + _NBUF_TR < _BPW_TR)
            def _():
                fire_in(g + _NBUF_TR, b)

        return carry

    lax.fori_loop(0, _BPW_TR // _NBUF_TR, outer, 0)
    for b in range(_NBUF_TR):
        g = _BPW_TR - _NBUF_TR + b
        pltpu.make_async_copy(obufs[b], out_slice(g), osems[b]).wait()

    @pl.when(wid < _TR_EXTRA)
    def _():
        blk = _BPW_TR * _NW + wid
        pltpu.sync_copy(tabT_hbm.at[:, pl.ds(blk * _BLK, _BLK)], ibufs[0])
        _tr_transpose_block(ibufs[0], obufs[0], iota, half, qbit,
                            _BLK // _LANES)
        pltpu.sync_copy(obufs[0],
                        tab2_hbm.at[pl.ds(blk * (_BLK // 2), _BLK // 2)])

    @pl.when(wid == _TR_EXTRA)
    def _():
        pltpu.sync_copy(tailp_hbm.at[pl.ds(_D // 2, _D // 2)],
                        tab2_hbm.at[pl.ds((_VOCAB - _D) // 2, _D // 2)])


def _tr_kernel(tabT_hbm, tailp_hbm, tab2_hbm, *refs):
    n = _NBUF_TR
    _tr_body(tabT_hbm, tailp_hbm, tab2_hbm, list(refs[0:n]),
             list(refs[n:2 * n]), list(refs[2 * n:3 * n]),
             list(refs[3 * n:4 * n]))


def _gather_body(tab2_hbm, idxT_hbm, pe_hbm, outT_hbm,
                 idx_all, pe_v, ihs, gbufs, tbufs, gsems, osems):
    wid = lax.axis_index("s") * _NC + lax.axis_index("c")
    b0 = wid * _BPW

    pltpu.sync_copy(idxT_hbm.at[:, pl.ds(b0, _BPW)], idx_all)
    pltpu.sync_copy(pe_hbm, pe_v)

    def fire_gather(s, b):
        for k in range(_KG):
            sl = pl.ds(k * _LANES, _LANES)
            ihs[b][sl] = lax.shift_right_logical(idx_all[s, sl], 1)
        pltpu.async_copy(tab2_hbm.at[ihs[b]], gbufs[b], gsems[b])

    def wait_gather(b):
        pltpu.make_async_copy(tab2_hbm.at[ihs[b]], gbufs[b], gsems[b]).wait()

    def out_slice(s):
        return outT_hbm.at[s, :, pl.ds(b0, _BPW)]

    for b in range(_NBUF):
        fire_gather(b, b)

    def outer(o, carry):
        for b in range(_NBUF):
            s = o * _NBUF + b
            wait_gather(b)

            @pl.when(o > 0)
            def _():
                pltpu.make_async_copy(tbufs[b], out_slice(s),
                                      osems[b]).wait()

            gbuf, tbuf = gbufs[b], tbufs[b]
            iota = lax.iota(jnp.int32, _LANES)
            colbase = []
            for k in range(_KG):
                sl = pl.ds(k * _LANES, _LANES)
                colbase.append((idx_all[s, sl] & 1) * _D)

            @plsc.parallel_loop(0, _D, 1, unroll=4)
            def do_t(t):
                dvec = ((iota + t) & (_LANES - 1)) + (t & ~(_LANES - 1))
                pe_vec = plsc.load_gather(pe_v, [s * _D + dvec])
                for k in range(_KG):
                    rows_k = iota + (k * _LANES)
                    vals = plsc.load_gather(gbuf, [rows_k, colbase[k] + dvec])
                    plsc.store_scatter(tbuf, [dvec, rows_k], vals + pe_vec)

            pltpu.async_copy(tbuf, out_slice(s), osems[b])

            @pl.when(s + _NBUF < _SEQ)
            def _():
                fire_gather(s + _NBUF, b)

        return carry

    lax.fori_loop(0, _NOUTER, outer, 0)

    for b in range(_NBUF):
        s = (_NOUTER - 1) * _NBUF + b
        pltpu.make_async_copy(tbufs[b], out_slice(s), osems[b]).wait()


def _gather_kernel(tab2_hbm, idxT_hbm, pe_hbm, outT_hbm,
                   idx_all, pe_v, i0_, i1_, g0, g1, t0, t1,
                   gs0, gs1, os0, os1):
    _gather_body(tab2_hbm, idxT_hbm, pe_hbm, outT_hbm, idx_all, pe_v,
                 [i0_, i1_], [g0, g1], [t0, t1], [gs0, gs1], [os0, os1])


@jax.jit
def _sc_embed(tabT, tailp, idxT, pe_flat):
    mesh = plsc.VectorSubcoreMesh(core_axis_name="c", subcore_axis_name="s")
    tab2 = pl.kernel(
        _tr_kernel,
        out_type=jax.ShapeDtypeStruct((_VOCAB // 2, _PAIR_W), jnp.float32),
        mesh=mesh,
        scratch_types=(
            [pltpu.VMEM((_D, _BLK), jnp.float32) for _ in range(_NBUF_TR)]
            + [pltpu.VMEM((_BLK // 2, _PAIR_W), jnp.float32)
               for _ in range(_NBUF_TR)]
            + [pltpu.SemaphoreType.DMA for _ in range(2 * _NBUF_TR)]
        ),
        compiler_params=pltpu.CompilerParams(needs_layout_passes=False),
    )(tabT, tailp)
    return pl.kernel(
        _gather_kernel,
        out_type=jax.ShapeDtypeStruct((_SEQ, _D, _BATCH), jnp.float32),
        mesh=mesh,
        scratch_types=(
            [pltpu.VMEM((_SEQ, _BPW), jnp.int32),
             pltpu.VMEM((_SEQ * _D,), jnp.float32)]
            + [pltpu.VMEM((_BPW,), jnp.int32) for _ in range(_NBUF)]
            + [pltpu.VMEM((_BPW, _PAIR_W), jnp.float32) for _ in range(_NBUF)]
            + [pltpu.VMEM((_D, _BPW), jnp.float32) for _ in range(_NBUF)]
            + [pltpu.SemaphoreType.DMA for _ in range(2 * _NBUF)]
        ),
        compiler_params=pltpu.CompilerParams(needs_layout_passes=False),
    )(tab2, idxT, pe_flat)


def kernel(indices, table, pos_enc):
    tabT = table.T
    tailp = table[-2 * _D:].reshape(_D, _PAIR_W)
    idxT = indices.T.astype(jnp.int32)
    pe_flat = pos_enc.reshape(-1)
    outT = _sc_embed(tabT, tailp, idxT, pe_flat)
    return outT.transpose(2, 0, 1)


# speedup vs baseline: 1.2164x; 1.0556x over previous
import jax
import jax.numpy as jnp
from jax import lax
from jax.experimental import pallas as pl
from jax.experimental.pallas import tpu as pltpu
from jax.experimental.pallas import tpu_sc as plsc

_VOCAB = 1000000
_SEQ = 200
_D = 64
_BATCH = 4096

_NC = 2
_NS = 16
_NW = _NC * _NS

_BPW = _BATCH // _NW
_NBUF = 2
_NOUTER = _SEQ // _NBUF
_LANES = 16
_PAIR_W = 2 * _D
_KG = _BPW // _LANES

_NBUF_TR = 4
_NBUF_G = 3
_SEQ_PIPE = _SEQ - _SEQ % _NBUF_G
_BLK = 128
_NFULL = _VOCAB // _BLK
_BPW_TR = _NFULL // _NW
_TR_EXTRA = _NFULL - _BPW_TR * _NW


def _tr_transpose_block(ibuf, obuf, iota, half, qbit, n_r0):
    @plsc.parallel_loop(0, _D, 1, unroll=4)
    def do_t(t):
        dvec = ((iota + t) & (_LANES - 1)) + (t & ~(_LANES - 1))
        colv = qbit + dvec
        for r0 in range(0, n_r0 * 8, 8):
            vals = plsc.load_gather(ibuf, [dvec, iota + 2 * r0])
            plsc.store_scatter(obuf, [half + r0, colv], vals)


def _tr_body(tabT_hbm, tailp_hbm, tab2_hbm, ibufs, obufs, isems, osems):
    wid = lax.axis_index("s") * _NC + lax.axis_index("c")
    b0 = wid * _BPW_TR
    iota = lax.iota(jnp.int32, _LANES)
    half = lax.shift_right_logical(iota, 1)
    qbit = (iota & 1) * _D

    def in_slice(g):
        off = pl.multiple_of((b0 + g) * _BLK, _BLK)
        return tabT_hbm.at[:, pl.ds(off, _BLK)]

    def fire_in(g, b):
        pltpu.async_copy(in_slice(g), ibufs[b], isems[b])

    def wait_in(g, b):
        pltpu.make_async_copy(in_slice(g), ibufs[b], isems[b]).wait()

    def out_slice(g):
        return tab2_hbm.at[pl.ds((b0 + g) * (_BLK // 2), _BLK // 2)]

    for b in range(_NBUF_TR):
        fire_in(b, b)

    def outer(o, carry):
        for b in range(_NBUF_TR):
            g = o * _NBUF_TR + b
            wait_in(g, b)

            @pl.when(o > 0)
            def _():
                pltpu.make_async_copy(obufs[b], out_slice(g), osems[b]).wait()

            _tr_transpose_block(ibufs[b], obufs[b], iota, half, qbit,
                                _BLK // _LANES)
            pltpu.async_copy(obufs[b], out_slice(g), osems[b])

            @pl.when(g + _NBUF_TR < _BPW_TR)
            def _():
                fire_in(g + _NBUF_TR, b)

        return carry

    lax.fori_loop(0, _BPW_TR // _NBUF_TR, outer, 0)
    for b in range(_NBUF_TR):
        g = _BPW_TR - _NBUF_TR + b
        pltpu.make_async_copy(obufs[b], out_slice(g), osems[b]).wait()

    @pl.when(wid < _TR_EXTRA)
    def _():
        blk = _BPW_TR * _NW + wid
        pltpu.sync_copy(tabT_hbm.at[:, pl.ds(blk * _BLK, _BLK)], ibufs[0])
        _tr_transpose_block(ibufs[0], obufs[0], iota, half, qbit,
                            _BLK // _LANES)
        pltpu.sync_copy(obufs[0],
                        tab2_hbm.at[pl.ds(blk * (_BLK // 2), _BLK // 2)])

    @pl.when(wid == _TR_EXTRA)
    def _():
        pltpu.sync_copy(tailp_hbm.at[pl.ds(_D // 2, _D // 2)],
                        tab2_hbm.at[pl.ds((_VOCAB - _D) // 2, _D // 2)])


def _tr_kernel(tabT_hbm, tailp_hbm, tab2_hbm, *refs):
    n = _NBUF_TR
    _tr_body(tabT_hbm, tailp_hbm, tab2_hbm, list(refs[0:n]),
             list(refs[n:2 * n]), list(refs[2 * n:3 * n]),
             list(refs[3 * n:4 * n]))


def _gather_body(tab2_hbm, idxT_hbm, pe_hbm, outT_hbm,
                 idx_all, pe_v, ihs, gbufs, tbufs, gsems, osems):
    wid = lax.axis_index("s") * _NC + lax.axis_index("c")
    b0 = wid * _BPW

    pltpu.sync_copy(idxT_hbm.at[:, pl.ds(b0, _BPW)], idx_all)
    pltpu.sync_copy(pe_hbm, pe_v)

    def fire_gather(s, b):
        for k in range(_KG):
            sl = pl.ds(k * _LANES, _LANES)
            ihs[b][sl] = lax.shift_right_logical(idx_all[s, sl], 1)
        pltpu.async_copy(tab2_hbm.at[ihs[b]], gbufs[b], gsems[b])

    def wait_gather(b):
        pltpu.make_async_copy(tab2_hbm.at[ihs[b]], gbufs[b], gsems[b]).wait()

    def out_slice(s):
        return outT_hbm.at[s, :, pl.ds(b0, _BPW)]

    def transpose_add(s, b):
        gbuf, tbuf = gbufs[b], tbufs[b]
        iota = lax.iota(jnp.int32, _LANES)
        colbase = []
        for k in range(_KG):
            sl = pl.ds(k * _LANES, _LANES)
            colbase.append((idx_all[s, sl] & 1) * _D)

        @plsc.parallel_loop(0, _D, 1, unroll=4)
        def do_t(t):
            dvec = ((iota + t) & (_LANES - 1)) + (t & ~(_LANES - 1))
            pe_vec = plsc.load_gather(pe_v, [s * _D + dvec])
            for k in range(_KG):
                rows_k = iota + (k * _LANES)
                vals = plsc.load_gather(gbuf, [rows_k, colbase[k] + dvec])
                plsc.store_scatter(tbuf, [dvec, rows_k], vals + pe_vec)

    for b in range(_NBUF_G):
        fire_gather(b, b)

    def outer(o, carry):
        for b in range(_NBUF_G):
            s = o * _NBUF_G + b
            wait_gather(b)

            @pl.when(o > 0)
            def _():
                pltpu.make_async_copy(tbufs[b], out_slice(s),
                                      osems[b]).wait()

            transpose_add(s, b)
            pltpu.async_copy(tbufs[b], out_slice(s), osems[b])

            @pl.when(s + _NBUF_G < _SEQ_PIPE)
            def _():
                fire_gather(s + _NBUF_G, b)

        return carry

    lax.fori_loop(0, _SEQ_PIPE // _NBUF_G, outer, 0)

    for b in range(_NBUF_G):
        s = _SEQ_PIPE - _NBUF_G + b
        pltpu.make_async_copy(tbufs[b], out_slice(s), osems[b]).wait()

    for s in range(_SEQ_PIPE, _SEQ):
        b = s - _SEQ_PIPE
        fire_gather(s, b)
        wait_gather(b)
        transpose_add(s, b)
        pltpu.sync_copy(tbufs[b], out_slice(s))


def _gather_kernel(tab2_hbm, idxT_hbm, pe_hbm, outT_hbm,
                   idx_all, pe_v, *refs):
    n = _NBUF_G
    _gather_body(tab2_hbm, idxT_hbm, pe_hbm, outT_hbm, idx_all, pe_v,
                 list(refs[0:n]), list(refs[n:2 * n]),
                 list(refs[2 * n:3 * n]), list(refs[3 * n:4 * n]),
                 list(refs[4 * n:5 * n]))


@jax.jit
def _sc_embed(tabT, tailp, idxT, pe_flat):
    mesh = plsc.VectorSubcoreMesh(core_axis_name="c", subcore_axis_name="s")
    tab2 = pl.kernel(
        _tr_kernel,
        out_type=jax.ShapeDtypeStruct((_VOCAB // 2, _PAIR_W), jnp.float32),
        mesh=mesh,
        scratch_types=(
            [pltpu.VMEM((_D, _BLK), jnp.float32) for _ in range(_NBUF_TR)]
            + [pltpu.VMEM((_BLK // 2, _PAIR_W), jnp.float32)
               for _ in range(_NBUF_TR)]
            + [pltpu.SemaphoreType.DMA for _ in range(2 * _NBUF_TR)]
        ),
        compiler_params=pltpu.CompilerParams(needs_layout_passes=False),
    )(tabT, tailp)
    return pl.kernel(
        _gather_kernel,
        out_type=jax.ShapeDtypeStruct((_SEQ, _D, _BATCH), jnp.float32),
        mesh=mesh,
        scratch_types=(
            [pltpu.VMEM((_SEQ, _BPW), jnp.int32),
             pltpu.VMEM((_SEQ * _D,), jnp.float32)]
            + [pltpu.VMEM((_BPW,), jnp.int32) for _ in range(_NBUF_G)]
            + [pltpu.VMEM((_BPW, _PAIR_W), jnp.float32)
               for _ in range(_NBUF_G)]
            + [pltpu.VMEM((_D, _BPW), jnp.float32) for _ in range(_NBUF_G)]
            + [pltpu.SemaphoreType.DMA for _ in range(2 * _NBUF_G)]
        ),
        compiler_params=pltpu.CompilerParams(needs_layout_passes=False),
    )(tab2, idxT, pe_flat)


def kernel(indices, table, pos_enc):
    tabT = table.T
    tailp = table[-2 * _D:].reshape(_D, _PAIR_W)
    idxT = indices.T.astype(jnp.int32)
    pe_flat = pos_enc.reshape(-1)
    outT = _sc_embed(tabT, tailp, idxT, pe_flat)
    return outT.transpose(2, 0, 1)
